# baseline (speedup 1.0000x reference)
import jax
import jax.numpy as jnp
from jax import lax
from jax.experimental import pallas as pl
from jax.experimental.pallas import tpu as pltpu
from jax.experimental.pallas import tpu_sc as plsc

L = 16
NCOMP = 64
NGROUP = NCOMP // L
NHALF = NCOMP // 2


def _worker_range(bs):
    info = plsc.get_sparse_core_info()
    nw = info.num_cores * info.num_subcores
    qpt = bs // nw
    wid = lax.axis_index("s") * info.num_cores + lax.axis_index("c")
    return qpt, wid * qpt


def _dist_body(x_hbm, neigh_hbm, sx_hbm, coef_hbm, w_hbm,
               neigh_tile, xrows0, xrows1, xq0, xq1, w_tile, coef_v,
               semx0, semx1, semq0, semq1):
    dim = x_hbm.shape[1]
    nchunk = dim // L
    qpt, base = _worker_range(x_hbm.shape[0])

    xrows = (xrows0, xrows1)
    xq = (xq0, xq1)
    semx = (semx0, semx1)
    semq = (semq0, semq1)

    pltpu.sync_copy(coef_hbm, coef_v)
    pltpu.sync_copy(neigh_hbm.at[pl.ds(base, qpt)], neigh_tile)
    coef = coef_v[...]
    iota = lax.iota(jnp.int32, L)
    zeros = jnp.zeros((L,), jnp.float32)

    def fetch_copies(slot, q):
        q = jnp.minimum(q, qpt - 1)
        return (
            pltpu.make_async_copy(sx_hbm.at[neigh_tile.at[q, pl.ds(0, NHALF)]],
                                  xrows[slot].at[pl.ds(0, NHALF)], semx[slot]),
            pltpu.make_async_copy(sx_hbm.at[neigh_tile.at[q,
                                                          pl.ds(NHALF, NHALF)]],
                                  xrows[slot].at[pl.ds(NHALF, NHALF)],
                                  semx[slot]),
            pltpu.make_async_copy(x_hbm.at[base + q], xq[slot], semq[slot]),
        )

    def start_fetch(slot, q):
        for cp in fetch_copies(slot, q):
            cp.start()

    def wait_fetch(slot, q):
        for cp in fetch_copies(slot, q):
            cp.wait()

    def compute(slot, q):
        xqs = [xq[slot][pl.ds(L * c, L)] for c in range(nchunk)]

        def g_body(g, tsum):
            d2v = zeros
            row0 = g * L
            for j in range(L):
                acc = zeros
                for c in range(nchunk // 2):
                    cv = xrows[slot][row0 + j, pl.ds(L * c, L)]
                    d = xqs[c] - cv
                    acc = acc + d * d
                s = jnp.sum(acc)
                d2v = jnp.where(iota == j, s, d2v)
            w = jnp.exp(d2v * coef)
            w_tile[q, pl.ds(g * L, L)] = w
            return tsum + jnp.sum(w)

        tsum = lax.fori_loop(0, NGROUP, g_body, jnp.float32(0.0))
        inv = 1.0 / (zeros + tsum + 1e-9)
        for g in range(NGROUP):
            w_tile[q, pl.ds(g * L, L)] = w_tile[q, pl.ds(g * L, L)] * inv

    start_fetch(0, 0)

    def q_body(qq, _):
        q0 = 2 * qq
        start_fetch(1, q0 + 1)
        wait_fetch(0, q0)
        compute(0, q0)
        start_fetch(0, q0 + 2)
        wait_fetch(1, q0 + 1)
        compute(1, q0 + 1)
        return 0

    lax.fori_loop(0, qpt // 2, q_body, 0)
    wait_fetch(0, qpt - 1)
    pltpu.sync_copy(w_tile, w_hbm.at[pl.ds(base, qpt)])


def _hist_body(w_hbm, neigh_hbm, lab_hbm, out_hbm,
               neigh_tile, w_tile, labs_tile, out_tile,
               semg0, semg1, semg2, semg3):
    qpt, base = _worker_range(w_hbm.shape[0])
    iota = lax.iota(jnp.int32, L)
    zeros = jnp.zeros((L,), jnp.float32)
    semg = (semg0, semg1, semg2, semg3)

    pltpu.sync_copy(neigh_hbm.at[pl.ds(base, qpt)], neigh_tile)
    pltpu.sync_copy(w_hbm.at[pl.ds(base, qpt)], w_tile)
    cps = [pltpu.make_async_copy(lab_hbm.at[neigh_tile.at[q]],
                                 labs_tile.at[q], semg[q % 4])
           for q in range(qpt)]
    for cp in cps:
        cp.start()
    for cp in cps:
        cp.wait()

    def q_body(q, _):
        for k in range(8):
            out_tile[q, pl.ds(L * k, L)] = zeros
        qvec = jnp.full((L,), q, jnp.int32)
        for g in range(NGROUP):
            wg = w_tile[q, pl.ds(L * g, L)]
            lg = labs_tile[q, pl.ds(L * g, L)]
            for l in range(L):
                plsc.addupdate_scatter(out_tile, [qvec, lg], wg,
                                       mask=iota == l)
        return 0

    lax.fori_loop(0, qpt, q_body, 0)
    pltpu.sync_copy(out_tile, out_hbm.at[pl.ds(base, qpt)])


_MESH = dict(core_axis_name="c", subcore_axis_name="s")


@jax.jit
def _sc_call(x_enc, neighbors, samples_x, labels, coef):
    bs, dim = x_enc.shape
    info = plsc.get_sparse_core_info()
    qpt = bs // (info.num_cores * info.num_subcores)
    params = pltpu.CompilerParams(needs_layout_passes=False)

    w_all = pl.kernel(
        _dist_body,
        out_type=jax.ShapeDtypeStruct((bs, NCOMP), jnp.float32),
        mesh=plsc.VectorSubcoreMesh(**_MESH),
        compiler_params=params,
        scratch_types=[
            pltpu.VMEM((qpt, NCOMP), jnp.int32),
            pltpu.VMEM((NCOMP, dim), jnp.float32),
            pltpu.VMEM((NCOMP, dim), jnp.float32),
            pltpu.VMEM((dim,), jnp.float32),
            pltpu.VMEM((dim,), jnp.float32),
            pltpu.VMEM((qpt, NCOMP), jnp.float32),
            pltpu.VMEM((L,), jnp.float32),
            pltpu.SemaphoreType.DMA,
            pltpu.SemaphoreType.DMA,
            pltpu.SemaphoreType.DMA,
            pltpu.SemaphoreType.DMA,
        ],
    )(x_enc, neighbors, samples_x, coef)

    out = pl.kernel(
        _hist_body,
        out_type=jax.ShapeDtypeStruct((bs, 128), jnp.float32),
        mesh=plsc.VectorSubcoreMesh(**_MESH),
        compiler_params=params,
        scratch_types=[
            pltpu.VMEM((qpt, NCOMP), jnp.int32),
            pltpu.VMEM((qpt, NCOMP), jnp.float32),
            pltpu.VMEM((qpt, NCOMP), jnp.int32),
            pltpu.VMEM((qpt, 128), jnp.float32),
            pltpu.SemaphoreType.DMA,
            pltpu.SemaphoreType.DMA,
            pltpu.SemaphoreType.DMA,
            pltpu.SemaphoreType.DMA,
        ],
    )(w_all, neighbors, labels)
    return out


def kernel(x_enc, neighbors, samples_x, samples_y, sigma):
    neighbors = neighbors.astype(jnp.int32)
    classes = samples_y.shape[1]
    labels = jnp.sum(
        samples_y * jnp.arange(classes, dtype=jnp.float32), axis=1
    ).astype(jnp.int32)
    sigma = sigma.astype(jnp.float32)
    coef = jnp.full((L,), -1.0, jnp.float32) / (sigma * sigma)
    out = _sc_call(x_enc, neighbors, samples_x, labels, coef)
    return out[:, :classes]


# speedup vs baseline: 1.0045x; 1.0045x over previous
import jax
import jax.numpy as jnp
from jax import lax
from jax.experimental import pallas as pl
from jax.experimental.pallas import tpu as pltpu
from jax.experimental.pallas import tpu_sc as plsc

L = 16
NCOMP = 64
NGROUP = NCOMP // L
NHALF = NCOMP // 2


def _worker_range(bs):
    info = plsc.get_sparse_core_info()
    nw = info.num_cores * info.num_subcores
    qpt = bs // nw
    wid = lax.axis_index("s") * info.num_cores + lax.axis_index("c")
    return qpt, wid * qpt


def _dist_body(x_hbm, neigh_hbm, sx_hbm, coef_hbm, w_hbm,
               neigh_tile, xrows0, xrows1, xq0, xq1, w_tile, coef_v,
               semx0, semx1, semq0, semq1):
    dim = x_hbm.shape[1]
    nchunk = dim // L
    qpt, base = _worker_range(x_hbm.shape[0])

    xrows = (xrows0, xrows1)
    xq = (xq0, xq1)
    semx = (semx0, semx1)
    semq = (semq0, semq1)

    pltpu.sync_copy(coef_hbm, coef_v)
    pltpu.sync_copy(neigh_hbm.at[pl.ds(base, qpt)], neigh_tile)
    coef = coef_v[...]
    iota = lax.iota(jnp.int32, L)
    zeros = jnp.zeros((L,), jnp.float32)

    def fetch_copies(slot, q):
        q = jnp.minimum(q, qpt - 1)
        return (
            pltpu.make_async_copy(sx_hbm.at[neigh_tile.at[q, pl.ds(0, NHALF)]],
                                  xrows[slot].at[pl.ds(0, NHALF)], semx[slot]),
            pltpu.make_async_copy(sx_hbm.at[neigh_tile.at[q,
                                                          pl.ds(NHALF, NHALF)]],
                                  xrows[slot].at[pl.ds(NHALF, NHALF)],
                                  semx[slot]),
            pltpu.make_async_copy(x_hbm.at[base + q], xq[slot], semq[slot]),
        )

    def start_fetch(slot, q):
        for cp in fetch_copies(slot, q):
            cp.start()

    def wait_fetch(slot, q):
        for cp in fetch_copies(slot, q):
            cp.wait()

    def compute(slot, q):
        xqs = [xq[slot][pl.ds(L * c, L)] for c in range(nchunk)]

        def g_body(g, tsum):
            d2v = zeros
            row0 = g * L
            for j in range(L):
                acc0 = zeros
                acc1 = zeros
                half = nchunk // 2
                for c in range(half):
                    cv0 = xrows[slot][row0 + j, pl.ds(L * c, L)]
                    d0 = xqs[c] - cv0
                    acc0 = acc0 + d0 * d0
                    cv1 = xrows[slot][row0 + j, pl.ds(L * (c + half), L)]
                    d1 = xqs[c + half] - cv1
                    acc1 = acc1 + d1 * d1
                s = jnp.sum(acc0 + acc1)
                d2v = jnp.where(iota == j, s, d2v)
            w = jnp.exp(d2v * coef)
            w_tile[q, pl.ds(g * L, L)] = w
            return tsum + jnp.sum(w)

        tsum = lax.fori_loop(0, NGROUP, g_body, jnp.float32(0.0))
        inv = 1.0 / (zeros + tsum + 1e-9)
        for g in range(NGROUP):
            w_tile[q, pl.ds(g * L, L)] = w_tile[q, pl.ds(g * L, L)] * inv

    start_fetch(0, 0)

    def q_body(qq, _):
        q0 = 2 * qq
        start_fetch(1, q0 + 1)
        wait_fetch(0, q0)
        compute(0, q0)
        start_fetch(0, q0 + 2)
        wait_fetch(1, q0 + 1)
        compute(1, q0 + 1)
        return 0

    lax.fori_loop(0, qpt // 2, q_body, 0)
    wait_fetch(0, qpt - 1)
    pltpu.sync_copy(w_tile, w_hbm.at[pl.ds(base, qpt)])


def _hist_body(w_hbm, neigh_hbm, lab_hbm, out_hbm,
               neigh_tile, w_tile, labs_tile, out_tile,
               semg0, semg1, semg2, semg3):
    qpt, base = _worker_range(w_hbm.shape[0])
    iota = lax.iota(jnp.int32, L)
    zeros = jnp.zeros((L,), jnp.float32)
    semg = (semg0, semg1, semg2, semg3)

    pltpu.sync_copy(neigh_hbm.at[pl.ds(base, qpt)], neigh_tile)
    pltpu.sync_copy(w_hbm.at[pl.ds(base, qpt)], w_tile)
    cps = [pltpu.make_async_copy(lab_hbm.at[neigh_tile.at[q]],
                                 labs_tile.at[q], semg[q % 4])
           for q in range(qpt)]
    for cp in cps:
        cp.start()
    for cp in cps:
        cp.wait()

    def q_body(q, _):
        for k in range(8):
            out_tile[q, pl.ds(L * k, L)] = zeros
        qvec = jnp.full((L,), q, jnp.int32)
        for g in range(NGROUP):
            wg = w_tile[q, pl.ds(L * g, L)]
            lg = labs_tile[q, pl.ds(L * g, L)]
            for l in range(L):
                plsc.addupdate_scatter(out_tile, [qvec, lg], wg,
                                       mask=iota == l)
        return 0

    lax.fori_loop(0, qpt, q_body, 0)
    pltpu.sync_copy(out_tile, out_hbm.at[pl.ds(base, qpt)])


_MESH = dict(core_axis_name="c", subcore_axis_name="s")


@jax.jit
def _sc_call(x_enc, neighbors, samples_x, labels, coef):
    bs, dim = x_enc.shape
    info = plsc.get_sparse_core_info()
    qpt = bs // (info.num_cores * info.num_subcores)
    params = pltpu.CompilerParams(needs_layout_passes=False)

    w_all = pl.kernel(
        _dist_body,
        out_type=jax.ShapeDtypeStruct((bs, NCOMP), jnp.float32),
        mesh=plsc.VectorSubcoreMesh(**_MESH),
        compiler_params=params,
        scratch_types=[
            pltpu.VMEM((qpt, NCOMP), jnp.int32),
            pltpu.VMEM((NCOMP, dim), jnp.float32),
            pltpu.VMEM((NCOMP, dim), jnp.float32),
            pltpu.VMEM((dim,), jnp.float32),
            pltpu.VMEM((dim,), jnp.float32),
            pltpu.VMEM((qpt, NCOMP), jnp.float32),
            pltpu.VMEM((L,), jnp.float32),
            pltpu.SemaphoreType.DMA,
            pltpu.SemaphoreType.DMA,
            pltpu.SemaphoreType.DMA,
            pltpu.SemaphoreType.DMA,
        ],
    )(x_enc, neighbors, samples_x, coef)

    out = pl.kernel(
        _hist_body,
        out_type=jax.ShapeDtypeStruct((bs, 128), jnp.float32),
        mesh=plsc.VectorSubcoreMesh(**_MESH),
        compiler_params=params,
        scratch_types=[
            pltpu.VMEM((qpt, NCOMP), jnp.int32),
            pltpu.VMEM((qpt, NCOMP), jnp.float32),
            pltpu.VMEM((qpt, NCOMP), jnp.int32),
            pltpu.VMEM((qpt, 128), jnp.float32),
            pltpu.SemaphoreType.DMA,
            pltpu.SemaphoreType.DMA,
            pltpu.SemaphoreType.DMA,
            pltpu.SemaphoreType.DMA,
        ],
    )(w_all, neighbors, labels)
    return out


def kernel(x_enc, neighbors, samples_x, samples_y, sigma):
    neighbors = neighbors.astype(jnp.int32)
    classes = samples_y.shape[1]
    labels = jnp.sum(
        samples_y * jnp.arange(classes, dtype=jnp.float32), axis=1
    ).astype(jnp.int32)
    sigma = sigma.astype(jnp.float32)
    coef = jnp.full((L,), -1.0, jnp.float32) / (sigma * sigma)
    out = _sc_call(x_enc, neighbors, samples_x, labels, coef)
    return out[:, :classes]
